# baseline (speedup 1.0000x reference)
import functools

import jax
import jax.numpy as jnp
from jax import lax
from jax.experimental import pallas as pl
from jax.experimental.pallas import tpu as pltpu
from jax.experimental.pallas import tpu_sc as plsc

N = 50000
E = 800000
HID = 64
HC = HID // 2

NSUB = 16
RPS_A = 3128
RPS_B = N - 15 * RPS_A

CHE_L = 400
CHE_0 = 1000

EPS_L = E // NSUB
NCH_L = EPS_L // CHE_L
NPAIR_L = NCH_L // 2

EPS_0 = E // (2 * NSUB)
NCH_0 = EPS_0 // CHE_0

F0 = 8

BLK = 5000
NG = N // BLK


def _slab_copy(s, src, dst):
  off = pl.multiple_of(s * RPS_A, 8)

  @pl.when(s < NSUB - 1)
  def _():
    pltpu.sync_copy(src.at[pl.ds(off, RPS_A)], dst.at[pl.ds(off, RPS_A)])

  @pl.when(s == NSUB - 1)
  def _():
    pltpu.sync_copy(src.at[pl.ds(15 * RPS_A, RPS_B)],
                    dst.at[pl.ds(15 * RPS_A, RPS_B)])


def _edge_loop(h_hbm, acc, src1d, dst1d, sidx, didx, dbufs, gsems, ssems,
               base_edge, n_chunks, che):
  g_cps = [None] * n_chunks
  s_cps = [None] * n_chunks
  for j in range(n_chunks):
    b = j % 2
    if j >= 2:
      s_cps[j - 2].wait()
    e0 = pl.multiple_of(base_edge + j * che, 8)
    pltpu.sync_copy(src1d.at[pl.ds(e0, che)], sidx[b])
    pltpu.sync_copy(dst1d.at[pl.ds(e0, che)], didx[b])
    g_cps[j] = pltpu.async_copy(h_hbm.at[sidx[b]], dbufs[b], gsems[b])
    if j >= 1:
      g_cps[j - 1].wait()
      s_cps[j - 1] = pltpu.async_copy(
          dbufs[1 - b], acc.at[didx[1 - b]], ssems[1 - b], add=True)
  g_cps[n_chunks - 1].wait()
  b = (n_chunks - 1) % 2
  s_cps[n_chunks - 1] = pltpu.async_copy(
      dbufs[b], acc.at[didx[b]], ssems[b], add=True)
  s_cps[n_chunks - 2].wait()
  s_cps[n_chunks - 1].wait()


def _edge_loop_pairs(h_hbm, acc, src1d, dst1d, sidx, didx, dbufs, gsems,
                     ssems, base_edge):
  def drain_scatter(b):
    pltpu.make_async_copy(dbufs[b], acc.at[didx[b]], ssems[b]).wait()

  def do_chunk(b, e0, wait_prev):
    if wait_prev:
      drain_scatter(b)
    pltpu.sync_copy(src1d.at[pl.ds(e0, CHE_L)], sidx[b])
    pltpu.sync_copy(dst1d.at[pl.ds(e0, CHE_L)], didx[b])
    return pltpu.async_copy(h_hbm.at[sidx[b]], dbufs[b], gsems[b])

  def pair(g, _):
    j0 = 2 * g
    e0 = pl.multiple_of(base_edge + j0 * CHE_L, 8)
    e1 = pl.multiple_of(base_edge + (j0 + 1) * CHE_L, 8)

    @pl.when(g >= 1)
    def _():
      drain_scatter(0)
    g0 = do_chunk(0, e0, False)

    @pl.when(g >= 1)
    def _():
      drain_scatter(1)
    g1 = do_chunk(1, e1, False)

    g0.wait()
    pltpu.async_copy(dbufs[0], acc.at[didx[0]], ssems[0], add=True)
    g1.wait()
    pltpu.async_copy(dbufs[1], acc.at[didx[1]], ssems[1], add=True)
    return 0

  lax.fori_loop(0, NPAIR_L, pair, 0)

  e_t = pl.multiple_of(base_edge + (NCH_L - 1) * CHE_L, 8)
  gt = do_chunk(0, e_t, True)
  drain_scatter(1)
  gt.wait()
  pltpu.async_copy(dbufs[0], acc.at[didx[0]], ssems[0], add=True)
  drain_scatter(0)


def _make_sc_agg_l():
  mesh = plsc.VectorSubcoreMesh(core_axis_name="c", subcore_axis_name="s")

  @functools.partial(
      pl.kernel,
      out_type=(
          jax.ShapeDtypeStruct((N, HC), jnp.float32),
          jax.ShapeDtypeStruct((N, HC), jnp.float32),
      ),
      mesh=mesh,
      scratch_types=(
          [pltpu.VMEM((CHE_L,), jnp.int32)] * 4
          + [pltpu.VMEM((CHE_L, HC), jnp.float32)] * 2
          + [pltpu.SemaphoreType.DMA] * 4
          + [pltpu.VMEM_SHARED((N, HC), jnp.float32)]
      ),
      compiler_params=pltpu.CompilerParams(use_tc_tiling_on_sc=False),
  )
  def sc_agg(h0, h1, src1d, dst1d, zer, out0, out1,
             si0, si1, di0, di1, db0, db1, gs0, gs1, ss0, ss1, acc):
    sidx, didx, dbufs = (si0, si1), (di0, di1), (db0, db1)
    gsems, ssems = (gs0, gs1), (ss0, ss1)
    c = lax.axis_index("c")
    s = lax.axis_index("s")
    _slab_copy(s, zer, acc)
    plsc.subcore_barrier()
    base = s * EPS_L

    @pl.when(c == 0)
    def _():
      _edge_loop_pairs(h0, acc, src1d, dst1d, sidx, didx, dbufs, gsems,
                       ssems, base)

    @pl.when(c == 1)
    def _():
      _edge_loop_pairs(h1, acc, src1d, dst1d, sidx, didx, dbufs, gsems,
                       ssems, base)

    plsc.subcore_barrier()

    @pl.when(c == 0)
    def _():
      _slab_copy(s, acc, out0)

    @pl.when(c == 1)
    def _():
      _slab_copy(s, acc, out1)

  return sc_agg


def _make_sc_agg_0():
  mesh = plsc.VectorSubcoreMesh(core_axis_name="c", subcore_axis_name="s")

  @functools.partial(
      pl.kernel,
      out_type=(
          jax.ShapeDtypeStruct((N, F0), jnp.float32),
          jax.ShapeDtypeStruct((N, F0), jnp.float32),
      ),
      mesh=mesh,
      scratch_types=(
          [pltpu.VMEM((CHE_0,), jnp.int32)] * 4
          + [pltpu.VMEM((CHE_0, F0), jnp.float32)] * 2
          + [pltpu.SemaphoreType.DMA] * 4
          + [pltpu.VMEM_SHARED((N, F0), jnp.float32)]
      ),
      compiler_params=pltpu.CompilerParams(use_tc_tiling_on_sc=False),
  )
  def sc_agg0(feat8, src1d, dst1d, zer, out0, out1,
              si0, si1, di0, di1, db0, db1, gs0, gs1, ss0, ss1, acc):
    sidx, didx, dbufs = (si0, si1), (di0, di1), (db0, db1)
    gsems, ssems = (gs0, gs1), (ss0, ss1)
    c = lax.axis_index("c")
    s = lax.axis_index("s")
    _slab_copy(s, zer, acc)
    plsc.subcore_barrier()
    wid = c * NSUB + s
    _edge_loop(feat8, acc, src1d, dst1d, sidx, didx, dbufs, gsems, ssems,
               wid * EPS_0, NCH_0, CHE_0)
    plsc.subcore_barrier()

    @pl.when(c == 0)
    def _():
      _slab_copy(s, acc, out0)

    @pl.when(c == 1)
    def _():
      _slab_copy(s, acc, out1)

  return sc_agg0


def _tc_stats(h0, h1, m2):
  def body(h0_ref, h1_ref, m_ref, means_ref, acc):
    i = pl.program_id(0)
    h = jnp.concatenate([h0_ref[...], h1_ref[...]], axis=1)
    _masked_stats(i, m_ref[...], h, acc, means_ref)

  return pl.pallas_call(
      body,
      grid=(NG,),
      in_specs=[
          pl.BlockSpec((BLK, HC), lambda i: (i, 0)),
          pl.BlockSpec((BLK, HC), lambda i: (i, 0)),
          pl.BlockSpec((BLK, 2), lambda i: (i, 0)),
      ],
      out_specs=pl.BlockSpec((3, HID), lambda i: (0, 0)),
      out_shape=jax.ShapeDtypeStruct((3, HID), jnp.float32),
      scratch_shapes=[pltpu.VMEM((3, HID + 1), jnp.float32)],
  )(h0, h1, m2)


def _masked_stats(i, m2, h, acc, means_ref):
  other = (1.0 - m2[:, 0:1]) * (1.0 - m2[:, 1:2])
  m3 = jnp.concatenate([m2, other], axis=1)
  hb = jnp.concatenate([h, jnp.ones((h.shape[0], 1), jnp.float32)], axis=1)
  p = lax.dot_general(m3, hb, (((0,), (0,)), ((), ())),
                      preferred_element_type=jnp.float32)

  @pl.when(i == 0)
  def _():
    acc[...] = p

  @pl.when(i > 0)
  def _():
    acc[...] = acc[...] + p

  @pl.when(i == NG - 1)
  def _():
    tot = acc[...]
    cnt = jnp.maximum(tot[:, HID:HID + 1], 1.0)
    means_ref[...] = tot[:, :HID] / cnt


def _tc_layer0(feat, p0, p1, W1, b1, W2, b2):
  def body(feat_ref, p0_ref, p1_ref, w1_ref, b1_ref, w2_ref, b2_ref,
           h0_ref, h1_ref):
    z = feat_ref[...] + p0_ref[:, 0:2] + p1_ref[:, 0:2]
    a = jnp.maximum(jnp.dot(z, w1_ref[...],
                            preferred_element_type=jnp.float32)
                    + b1_ref[...], 0.0)
    h = jnp.maximum(jnp.dot(a, w2_ref[...],
                            preferred_element_type=jnp.float32)
                    + b2_ref[...], 0.0)
    h0_ref[...] = h[:, :HC]
    h1_ref[...] = h[:, HC:]

  return pl.pallas_call(
      body,
      grid=(NG,),
      in_specs=[
          pl.BlockSpec((BLK, 2), lambda i: (i, 0)),
          pl.BlockSpec((BLK, F0), lambda i: (i, 0)),
          pl.BlockSpec((BLK, F0), lambda i: (i, 0)),
          pl.BlockSpec((2, HID), lambda i: (0, 0)),
          pl.BlockSpec((1, HID), lambda i: (0, 0)),
          pl.BlockSpec((HID, HID), lambda i: (0, 0)),
          pl.BlockSpec((1, HID), lambda i: (0, 0)),
      ],
      out_specs=[
          pl.BlockSpec((BLK, HC), lambda i: (i, 0)),
          pl.BlockSpec((BLK, HC), lambda i: (i, 0)),
      ],
      out_shape=[
          jax.ShapeDtypeStruct((N, HC), jnp.float32),
          jax.ShapeDtypeStruct((N, HC), jnp.float32),
      ],
  )(feat, p0, p1, W1, b1, W2, b2)


def _tc_layer(h0, h1, a0, a1, W1, b1, W2, b2):
  def body(h0_ref, h1_ref, a0_ref, a1_ref,
           w1_ref, b1_ref, w2_ref, b2_ref, o0_ref, o1_ref):
    z = jnp.concatenate(
        [h0_ref[...] + a0_ref[...], h1_ref[...] + a1_ref[...]], axis=1)
    a = jnp.maximum(jnp.dot(z, w1_ref[...],
                            preferred_element_type=jnp.float32)
                    + b1_ref[...], 0.0)
    h = jnp.maximum(jnp.dot(a, w2_ref[...],
                            preferred_element_type=jnp.float32)
                    + b2_ref[...], 0.0)
    o0_ref[...] = h[:, :HC]
    o1_ref[...] = h[:, HC:]

  return pl.pallas_call(
      body,
      grid=(NG,),
      in_specs=[
          pl.BlockSpec((BLK, HC), lambda i: (i, 0)),
          pl.BlockSpec((BLK, HC), lambda i: (i, 0)),
          pl.BlockSpec((BLK, HC), lambda i: (i, 0)),
          pl.BlockSpec((BLK, HC), lambda i: (i, 0)),
          pl.BlockSpec((HID, HID), lambda i: (0, 0)),
          pl.BlockSpec((1, HID), lambda i: (0, 0)),
          pl.BlockSpec((HID, HID), lambda i: (0, 0)),
          pl.BlockSpec((1, HID), lambda i: (0, 0)),
      ],
      out_specs=[
          pl.BlockSpec((BLK, HC), lambda i: (i, 0)),
          pl.BlockSpec((BLK, HC), lambda i: (i, 0)),
      ],
      out_shape=[
          jax.ShapeDtypeStruct((N, HC), jnp.float32),
          jax.ShapeDtypeStruct((N, HC), jnp.float32),
      ],
  )(h0, h1, a0, a1, W1, b1, W2, b2)


_sc_agg_l = _make_sc_agg_l()
_sc_agg_0 = _make_sc_agg_0()


def kernel(feat, edge_index, u_mask, v_mask, W1_0, b1_0, W2_0, b2_0,
           W1_1, b1_1, W2_1, b2_1, W1_2, b1_2, W2_2, b2_2):
  src1d = edge_index[0]
  dst1d = edge_index[1]
  feat8 = jnp.pad(feat, ((0, 0), (0, F0 - feat.shape[1])))
  m2 = jnp.stack([u_mask, v_mask], axis=1).astype(jnp.float32)
  zer8 = jnp.zeros((N, F0), jnp.float32)
  zer32 = jnp.zeros((N, HC), jnp.float32)
  b1_0r, b2_0r = b1_0.reshape(1, HID), b2_0.reshape(1, HID)
  b1_1r, b2_1r = b1_1.reshape(1, HID), b2_1.reshape(1, HID)
  b1_2r, b2_2r = b1_2.reshape(1, HID), b2_2.reshape(1, HID)

  p0, p1 = _sc_agg_0(feat8, src1d, dst1d, zer8)
  h0, h1 = _tc_layer0(feat, p0, p1, W1_0, b1_0r, W2_0, b2_0r)

  a0, a1 = _sc_agg_l(h0, h1, src1d, dst1d, zer32)
  mA = _tc_stats(h0, h1, m2)
  g0, g1 = _tc_layer(h0, h1, a0, a1, W1_1, b1_1r, W2_1, b2_1r)

  c0, c1 = _sc_agg_l(g0, g1, src1d, dst1d, zer32)
  mB = _tc_stats(g0, g1, m2)
  f0, f1 = _tc_layer(g0, g1, c0, c1, W1_2, b1_2r, W2_2, b2_2r)

  mC = _tc_stats(f0, f1, m2)

  return jnp.concatenate([mA.reshape(-1), mB.reshape(-1), mC.reshape(-1)])


# speedup vs baseline: 1.0416x; 1.0416x over previous
import functools

import jax
import jax.numpy as jnp
from jax import lax
from jax.experimental import pallas as pl
from jax.experimental.pallas import tpu as pltpu
from jax.experimental.pallas import tpu_sc as plsc

N = 50000
E = 800000
HID = 64
HC = HID // 2

NSUB = 16
RPS_A = 3128
RPS_B = N - 15 * RPS_A

CHE_L = 400
CHE_0 = 1000

EPS_L = E // NSUB
NCH_L = EPS_L // CHE_L
NPAIR_L = NCH_L // 2

EPS_0 = E // (2 * NSUB)
NCH_0 = EPS_0 // CHE_0

F0 = 8

BLK = 5000
NG = N // BLK


def _slab_copy(s, src, dst):
  off = pl.multiple_of(s * RPS_A, 8)

  @pl.when(s < NSUB - 1)
  def _():
    pltpu.sync_copy(src.at[pl.ds(off, RPS_A)], dst.at[pl.ds(off, RPS_A)])

  @pl.when(s == NSUB - 1)
  def _():
    pltpu.sync_copy(src.at[pl.ds(15 * RPS_A, RPS_B)],
                    dst.at[pl.ds(15 * RPS_A, RPS_B)])


def _edge_loop(h_hbm, acc, src1d, dst1d, sidx, didx, dbufs, gsems, ssems,
               base_edge, n_chunks, che):
  g_cps = [None] * n_chunks
  s_cps = [None] * n_chunks
  for j in range(n_chunks):
    b = j % 2
    if j >= 2:
      s_cps[j - 2].wait()
    e0 = pl.multiple_of(base_edge + j * che, 8)
    pltpu.sync_copy(src1d.at[pl.ds(e0, che)], sidx[b])
    pltpu.sync_copy(dst1d.at[pl.ds(e0, che)], didx[b])
    g_cps[j] = pltpu.async_copy(h_hbm.at[sidx[b]], dbufs[b], gsems[b])
    if j >= 1:
      g_cps[j - 1].wait()
      s_cps[j - 1] = pltpu.async_copy(
          dbufs[1 - b], acc.at[didx[1 - b]], ssems[1 - b], add=True)
  g_cps[n_chunks - 1].wait()
  b = (n_chunks - 1) % 2
  s_cps[n_chunks - 1] = pltpu.async_copy(
      dbufs[b], acc.at[didx[b]], ssems[b], add=True)
  s_cps[n_chunks - 2].wait()
  s_cps[n_chunks - 1].wait()


def _edge_loop_pairs(h_hbm, acc, src1d, dst1d, sidx, didx, dbufs, gsems,
                     ssems, base_edge):
  def drain_scatter(b):
    pltpu.make_async_copy(dbufs[b], acc.at[didx[b]], ssems[b]).wait()

  def do_chunk(b, e0, wait_prev):
    if wait_prev:
      drain_scatter(b)
    pltpu.sync_copy(src1d.at[pl.ds(e0, CHE_L)], sidx[b])
    pltpu.sync_copy(dst1d.at[pl.ds(e0, CHE_L)], didx[b])
    return pltpu.async_copy(h_hbm.at[sidx[b]], dbufs[b], gsems[b])

  def pair(g, _):
    j0 = 2 * g
    e0 = pl.multiple_of(base_edge + j0 * CHE_L, 8)
    e1 = pl.multiple_of(base_edge + (j0 + 1) * CHE_L, 8)

    @pl.when(g >= 1)
    def _():
      drain_scatter(0)
    g0 = do_chunk(0, e0, False)

    @pl.when(g >= 1)
    def _():
      drain_scatter(1)
    g1 = do_chunk(1, e1, False)

    g0.wait()
    pltpu.async_copy(dbufs[0], acc.at[didx[0]], ssems[0], add=True)
    g1.wait()
    pltpu.async_copy(dbufs[1], acc.at[didx[1]], ssems[1], add=True)
    return 0

  lax.fori_loop(0, NPAIR_L, pair, 0)

  e_t = pl.multiple_of(base_edge + (NCH_L - 1) * CHE_L, 8)
  gt = do_chunk(0, e_t, True)
  drain_scatter(1)
  gt.wait()
  pltpu.async_copy(dbufs[0], acc.at[didx[0]], ssems[0], add=True)
  drain_scatter(0)


def _make_sc_agg_l():
  mesh = plsc.VectorSubcoreMesh(core_axis_name="c", subcore_axis_name="s")

  @functools.partial(
      pl.kernel,
      out_type=(
          jax.ShapeDtypeStruct((N, HC), jnp.float32),
          jax.ShapeDtypeStruct((N, HC), jnp.float32),
      ),
      mesh=mesh,
      scratch_types=(
          [pltpu.VMEM((CHE_L,), jnp.int32)] * 4
          + [pltpu.VMEM((CHE_L, HC), jnp.float32)] * 2
          + [pltpu.SemaphoreType.DMA] * 4
          + [pltpu.VMEM_SHARED((N, HC), jnp.float32)]
      ),
      compiler_params=pltpu.CompilerParams(use_tc_tiling_on_sc=False),
  )
  def sc_agg(h0, h1, src1d, dst1d, zer, out0, out1,
             si0, si1, di0, di1, db0, db1, gs0, gs1, ss0, ss1, acc):
    sidx, didx, dbufs = (si0, si1), (di0, di1), (db0, db1)
    gsems, ssems = (gs0, gs1), (ss0, ss1)
    c = lax.axis_index("c")
    s = lax.axis_index("s")
    _slab_copy(s, zer, acc)
    plsc.subcore_barrier()
    base = s * EPS_L

    @pl.when(c == 0)
    def _():
      _edge_loop_pairs(h0, acc, src1d, dst1d, sidx, didx, dbufs, gsems,
                       ssems, base)

    @pl.when(c == 1)
    def _():
      _edge_loop_pairs(h1, acc, src1d, dst1d, sidx, didx, dbufs, gsems,
                       ssems, base)

    plsc.subcore_barrier()

    @pl.when(c == 0)
    def _():
      _slab_copy(s, acc, out0)

    @pl.when(c == 1)
    def _():
      _slab_copy(s, acc, out1)

  return sc_agg


def _make_sc_agg_0():
  mesh = plsc.VectorSubcoreMesh(core_axis_name="c", subcore_axis_name="s")

  @functools.partial(
      pl.kernel,
      out_type=(
          jax.ShapeDtypeStruct((N, F0), jnp.float32),
          jax.ShapeDtypeStruct((N, F0), jnp.float32),
      ),
      mesh=mesh,
      scratch_types=(
          [pltpu.VMEM((CHE_0,), jnp.int32)] * 4
          + [pltpu.VMEM((CHE_0, F0), jnp.float32)] * 2
          + [pltpu.SemaphoreType.DMA] * 4
          + [pltpu.VMEM_SHARED((N, F0), jnp.float32)]
      ),
      compiler_params=pltpu.CompilerParams(use_tc_tiling_on_sc=False),
  )
  def sc_agg0(feat8, src1d, dst1d, zer, out0, out1,
              si0, si1, di0, di1, db0, db1, gs0, gs1, ss0, ss1, acc):
    sidx, didx, dbufs = (si0, si1), (di0, di1), (db0, db1)
    gsems, ssems = (gs0, gs1), (ss0, ss1)
    c = lax.axis_index("c")
    s = lax.axis_index("s")
    _slab_copy(s, zer, acc)
    plsc.subcore_barrier()
    wid = c * NSUB + s
    _edge_loop(feat8, acc, src1d, dst1d, sidx, didx, dbufs, gsems, ssems,
               wid * EPS_0, NCH_0, CHE_0)
    plsc.subcore_barrier()

    @pl.when(c == 0)
    def _():
      _slab_copy(s, acc, out0)

    @pl.when(c == 1)
    def _():
      _slab_copy(s, acc, out1)

  return sc_agg0


def _masked_stats(i, m2, h, acc, means_ref):
  other = (1.0 - m2[:, 0:1]) * (1.0 - m2[:, 1:2])
  m3 = jnp.concatenate([m2, other], axis=1)
  hb = jnp.concatenate([h, jnp.ones((h.shape[0], 1), jnp.float32)], axis=1)
  p = lax.dot_general(m3, hb, (((0,), (0,)), ((), ())),
                      preferred_element_type=jnp.float32)

  @pl.when(i == 0)
  def _():
    acc[...] = p

  @pl.when(i > 0)
  def _():
    acc[...] = acc[...] + p

  @pl.when(i == NG - 1)
  def _():
    tot = acc[...]
    cnt = jnp.maximum(tot[:, HID:HID + 1], 1.0)
    means_ref[...] = tot[:, :HID] / cnt


def _tc_layer0(feat, p0, p1, m2, W1, b1, W2, b2):
  def body(feat_ref, p0_ref, p1_ref, m_ref, w1_ref, b1_ref, w2_ref, b2_ref,
           h0_ref, h1_ref, means_ref, acc):
    i = pl.program_id(0)
    z = feat_ref[...] + p0_ref[:, 0:2] + p1_ref[:, 0:2]
    a = jnp.maximum(jnp.dot(z, w1_ref[...],
                            preferred_element_type=jnp.float32)
                    + b1_ref[...], 0.0)
    h = jnp.maximum(jnp.dot(a, w2_ref[...],
                            preferred_element_type=jnp.float32)
                    + b2_ref[...], 0.0)
    h0_ref[...] = h[:, :HC]
    h1_ref[...] = h[:, HC:]
    _masked_stats(i, m_ref[...], h, acc, means_ref)

  return pl.pallas_call(
      body,
      grid=(NG,),
      in_specs=[
          pl.BlockSpec((BLK, 2), lambda i: (i, 0)),
          pl.BlockSpec((BLK, F0), lambda i: (i, 0)),
          pl.BlockSpec((BLK, F0), lambda i: (i, 0)),
          pl.BlockSpec((BLK, 2), lambda i: (i, 0)),
          pl.BlockSpec((2, HID), lambda i: (0, 0)),
          pl.BlockSpec((1, HID), lambda i: (0, 0)),
          pl.BlockSpec((HID, HID), lambda i: (0, 0)),
          pl.BlockSpec((1, HID), lambda i: (0, 0)),
      ],
      out_specs=[
          pl.BlockSpec((BLK, HC), lambda i: (i, 0)),
          pl.BlockSpec((BLK, HC), lambda i: (i, 0)),
          pl.BlockSpec((3, HID), lambda i: (0, 0)),
      ],
      out_shape=[
          jax.ShapeDtypeStruct((N, HC), jnp.float32),
          jax.ShapeDtypeStruct((N, HC), jnp.float32),
          jax.ShapeDtypeStruct((3, HID), jnp.float32),
      ],
      scratch_shapes=[pltpu.VMEM((3, HID + 1), jnp.float32)],
  )(feat, p0, p1, m2, W1, b1, W2, b2)


def _tc_layer(h0, h1, a0, a1, m2, W1, b1, W2, b2, write_h):
  def body(h0_ref, h1_ref, a0_ref, a1_ref, m_ref,
           w1_ref, b1_ref, w2_ref, b2_ref, *outs):
    if write_h:
      o0_ref, o1_ref, means_ref, acc = outs
    else:
      means_ref, acc = outs
    i = pl.program_id(0)
    z = jnp.concatenate(
        [h0_ref[...] + a0_ref[...], h1_ref[...] + a1_ref[...]], axis=1)
    a = jnp.maximum(jnp.dot(z, w1_ref[...],
                            preferred_element_type=jnp.float32)
                    + b1_ref[...], 0.0)
    h = jnp.maximum(jnp.dot(a, w2_ref[...],
                            preferred_element_type=jnp.float32)
                    + b2_ref[...], 0.0)
    if write_h:
      o0_ref[...] = h[:, :HC]
      o1_ref[...] = h[:, HC:]
    _masked_stats(i, m_ref[...], h, acc, means_ref)

  out_specs = []
  out_shape = []
  if write_h:
    out_specs += [pl.BlockSpec((BLK, HC), lambda i: (i, 0)),
                  pl.BlockSpec((BLK, HC), lambda i: (i, 0))]
    out_shape += [jax.ShapeDtypeStruct((N, HC), jnp.float32),
                  jax.ShapeDtypeStruct((N, HC), jnp.float32)]
  out_specs.append(pl.BlockSpec((3, HID), lambda i: (0, 0)))
  out_shape.append(jax.ShapeDtypeStruct((3, HID), jnp.float32))

  return pl.pallas_call(
      body,
      grid=(NG,),
      in_specs=[
          pl.BlockSpec((BLK, HC), lambda i: (i, 0)),
          pl.BlockSpec((BLK, HC), lambda i: (i, 0)),
          pl.BlockSpec((BLK, HC), lambda i: (i, 0)),
          pl.BlockSpec((BLK, HC), lambda i: (i, 0)),
          pl.BlockSpec((BLK, 2), lambda i: (i, 0)),
          pl.BlockSpec((HID, HID), lambda i: (0, 0)),
          pl.BlockSpec((1, HID), lambda i: (0, 0)),
          pl.BlockSpec((HID, HID), lambda i: (0, 0)),
          pl.BlockSpec((1, HID), lambda i: (0, 0)),
      ],
      out_specs=out_specs,
      out_shape=out_shape,
      scratch_shapes=[pltpu.VMEM((3, HID + 1), jnp.float32)],
  )(h0, h1, a0, a1, m2, W1, b1, W2, b2)


_sc_agg_l = _make_sc_agg_l()
_sc_agg_0 = _make_sc_agg_0()


def kernel(feat, edge_index, u_mask, v_mask, W1_0, b1_0, W2_0, b2_0,
           W1_1, b1_1, W2_1, b2_1, W1_2, b1_2, W2_2, b2_2):
  src1d = edge_index[0]
  dst1d = edge_index[1]
  feat8 = jnp.pad(feat, ((0, 0), (0, F0 - feat.shape[1])))
  m2 = jnp.stack([u_mask, v_mask], axis=1).astype(jnp.float32)
  zer8 = jnp.zeros((N, F0), jnp.float32)
  zer32 = jnp.zeros((N, HC), jnp.float32)
  b1_0r, b2_0r = b1_0.reshape(1, HID), b2_0.reshape(1, HID)
  b1_1r, b2_1r = b1_1.reshape(1, HID), b2_1.reshape(1, HID)
  b1_2r, b2_2r = b1_2.reshape(1, HID), b2_2.reshape(1, HID)

  p0, p1 = _sc_agg_0(feat8, src1d, dst1d, zer8)
  h0, h1, mA = _tc_layer0(feat, p0, p1, m2, W1_0, b1_0r, W2_0, b2_0r)

  a0, a1 = _sc_agg_l(h0, h1, src1d, dst1d, zer32)
  g0, g1, mB = _tc_layer(h0, h1, a0, a1, m2, W1_1, b1_1r, W2_1, b2_1r, True)

  c0, c1 = _sc_agg_l(g0, g1, src1d, dst1d, zer32)
  (mC,) = [_tc_layer(g0, g1, c0, c1, m2, W1_2, b1_2r, W2_2, b2_2r, False)[-1]]

  return jnp.concatenate([mA.reshape(-1), mB.reshape(-1), mC.reshape(-1)])
